# Initial kernel scaffold; baseline (speedup 1.0000x reference)
import jax
import jax.numpy as jnp
from jax.experimental import pallas as pl


def kernel(pred, target, label_weight):
    raise NotImplementedError("write your pallas kernel here")


# speedup vs baseline: 27.6133x; 27.6133x over previous
import functools

import numpy as np
import jax
import jax.numpy as jnp
from jax import lax
from jax.experimental import pallas as pl
from jax.experimental.pallas import tpu as pltpu
from jax.experimental.pallas import tpu_sc as plsc

NC = 2
NS = 16
L = 16
NW = NC * NS
BINS = 10
CHUNK = 16384
UNROLL = 4
THRESH = float(np.float32(np.float32(1.0) + np.float32(1e-6)))


def _wid():
    return lax.axis_index("s") * NC + lax.axis_index("c")


def _mesh():
    return plsc.VectorSubcoreMesh(
        core_axis_name="c", subcore_axis_name="s", num_cores=NC, num_subcores=NS
    )


def _stream_loop(p_hbm, t_hbm, w_hbm, bufs0, bufs1, sem0, sem1, nchunk, compute, carry):
    base = _wid() * (nchunk * CHUNK)

    def start(bufs, sem, k):
        off = base + k * CHUNK
        pltpu.async_copy(p_hbm.at[pl.ds(off, CHUNK)], bufs[0], sem)
        pltpu.async_copy(t_hbm.at[pl.ds(off, CHUNK)], bufs[1], sem)
        pltpu.async_copy(w_hbm.at[pl.ds(off, CHUNK)], bufs[2], sem)

    def wait(bufs, sem, k):
        off = base + k * CHUNK
        pltpu.make_async_copy(p_hbm.at[pl.ds(off, CHUNK)], bufs[0], sem).wait()
        pltpu.make_async_copy(t_hbm.at[pl.ds(off, CHUNK)], bufs[1], sem).wait()
        pltpu.make_async_copy(w_hbm.at[pl.ds(off, CHUNK)], bufs[2], sem).wait()

    start(bufs0, sem0, 0)

    def outer(k, carry):
        start(bufs1, sem1, 2 * k + 1)
        wait(bufs0, sem0, 2 * k)
        carry = compute(bufs0, carry)
        start(bufs0, sem0, 2 * k + 2)
        wait(bufs1, sem1, 2 * k + 1)
        carry = compute(bufs1, carry)
        return carry

    carry = lax.fori_loop(0, nchunk // 2 - 1, outer, carry)
    start(bufs1, sem1, nchunk - 1)
    wait(bufs0, sem0, nchunk - 2)
    carry = compute(bufs0, carry)
    wait(bufs1, sem1, nchunk - 1)
    carry = compute(bufs1, carry)
    return carry


def _pass1_body(nchunk, p_hbm, t_hbm, w_hbm, out_hbm,
                pb0, tb0, wb0, pb1, tb1, wb1, accb, sem0, sem1):
    def compute(bufs, acc):
        pb, tb, wb = bufs

        def inner(j, acc):
            a_i, a_p, a_t, a_v = acc
            for u in range(UNROLL):
                o = j * (L * UNROLL) + u * L
                p = pb[pl.ds(o, L)]
                t = tb[pl.ds(o, L)]
                w = wb[pl.ds(o, L)]
                a_i = a_i + p * t
                a_p = a_p + p
                a_t = a_t + t
                a_v = a_v + jnp.where(w > 0.0, 1.0, 0.0).astype(jnp.float32)
            return (a_i, a_p, a_t, a_v)

        return lax.fori_loop(0, CHUNK // (L * UNROLL), inner, acc)

    z = jnp.zeros((L,), jnp.float32)
    acc = _stream_loop(p_hbm, t_hbm, w_hbm, (pb0, tb0, wb0), (pb1, tb1, wb1),
                       sem0, sem1, nchunk, compute, (z, z, z, z))
    for i in range(4):
        accb[i, :] = acc[i]
    pltpu.sync_copy(accb, out_hbm.at[_wid()])


def _pass2_body(nchunk, p_hbm, t_hbm, w_hbm, c_hbm, out_hbm,
                pb0, tb0, wb0, pb1, tb1, wb1, cb, hist, sem0, sem1):
    pltpu.sync_copy(c_hbm, cb)
    c = cb[...]
    for i in range(2 * BINS):
        hist[pl.ds(i * L, L)] = jnp.zeros((L,), jnp.float32)
    lane = lax.iota(jnp.int32, L)
    ones = jnp.ones((L,), jnp.float32)

    def compute(bufs, carry):
        pb, tb, wb = bufs

        def inner(j, carry):
            for u in range(UNROLL):
                o = j * (L * UNROLL) + u * L
                p = pb[pl.ds(o, L)]
                t = tb[pl.ds(o, L)]
                w = wb[pl.ds(o, L)]
                g = jnp.abs(c * p - t)
                b = jnp.minimum((g * 10.0).astype(jnp.int32), BINS - 1)
                m = (g < THRESH) & (w > 0.0)
                slot = b * L + lane
                plsc.addupdate_scatter(hist, [slot], ones, mask=m)
                plsc.addupdate_scatter(hist, [slot + BINS * L], p * t, mask=m)
            return carry

        return lax.fori_loop(0, CHUNK // (L * UNROLL), inner, carry)

    _stream_loop(p_hbm, t_hbm, w_hbm, (pb0, tb0, wb0), (pb1, tb1, wb1),
                 sem0, sem1, nchunk, compute, 0)
    pltpu.sync_copy(hist, out_hbm.at[_wid()])


@functools.cache
def _build(n):
    assert n % (NW * CHUNK * 2) == 0, n
    nchunk = n // (NW * CHUNK)
    stream_bufs = [pltpu.VMEM((CHUNK,), jnp.float32) for _ in range(6)]

    params = pltpu.CompilerParams(needs_layout_passes=False)
    pass1 = pl.kernel(
        functools.partial(_pass1_body, nchunk),
        out_type=jax.ShapeDtypeStruct((NW, 4, L), jnp.float32),
        mesh=_mesh(),
        compiler_params=params,
        scratch_types=stream_bufs
        + [pltpu.VMEM((4, L), jnp.float32),
           pltpu.SemaphoreType.DMA, pltpu.SemaphoreType.DMA],
    )
    pass2 = pl.kernel(
        functools.partial(_pass2_body, nchunk),
        out_type=jax.ShapeDtypeStruct((NW, 2 * BINS * L), jnp.float32),
        mesh=_mesh(),
        compiler_params=params,
        scratch_types=stream_bufs
        + [pltpu.VMEM((L,), jnp.float32),
           pltpu.VMEM((2 * BINS * L,), jnp.float32),
           pltpu.SemaphoreType.DMA, pltpu.SemaphoreType.DMA],
    )
    return pass1, pass2


def kernel(pred, target, label_weight):
    p = pred.reshape(-1)
    t = target.reshape(-1).astype(jnp.float32)
    w = label_weight.reshape(-1).astype(jnp.float32)
    pass1, pass2 = _build(p.size)

    part1 = pass1(p, t, w)
    s = jnp.sum(part1, axis=(0, 2))
    i_sum, sum_p, sum_t, n_valid = s[0], s[1], s[2], s[3]
    big_s = sum_p + sum_t
    tot = jnp.maximum(n_valid, 1.0)
    cvec = jnp.full((L,), 2.0 * i_sum / big_s, jnp.float32)

    part2 = pass2(p, t, w, cvec)
    h = jnp.sum(part2.reshape(NW, 2 * BINS, L), axis=(0, 2))
    counts = h[:BINS]
    ptb = h[BINS:]
    n = jnp.sum((counts > 0).astype(jnp.float32))
    contrib = jnp.where(counts > 0, tot / jnp.maximum(counts, 1.0), 0.0) * ptb
    loss = 1.0 - (2.0 * jnp.sum(contrib) / jnp.maximum(n, 1.0)) / big_s
    return loss


# speedup vs baseline: 95.7705x; 3.4683x over previous
import functools

import numpy as np
import jax
import jax.numpy as jnp
from jax import lax
from jax.experimental import pallas as pl
from jax.experimental.pallas import tpu as pltpu
from jax.experimental.pallas import tpu_sc as plsc

NC = 2
NS = 16
L = 16
NW = NC * NS
BINS = 10
COLS = 1024
CROWS = 16
CHUNK = CROWS * COLS
UNROLL = 4
THRESH = float(np.float32(np.float32(1.0) + np.float32(1e-6)))


def _wid():
    return lax.axis_index("s") * NC + lax.axis_index("c")


def _mesh():
    return plsc.VectorSubcoreMesh(
        core_axis_name="c", subcore_axis_name="s", num_cores=NC, num_subcores=NS
    )


def _stream_loop(p_hbm, t_hbm, w_hbm, bufs0, bufs1, sem0, sem1, nchunk, compute, carry):
    row0 = _wid() * (nchunk * CROWS)

    def start(bufs, sem, k):
        r = row0 + k * CROWS
        pltpu.async_copy(p_hbm.at[pl.ds(r, CROWS), :], bufs[0], sem)
        pltpu.async_copy(t_hbm.at[pl.ds(r, CROWS), :], bufs[1], sem)
        pltpu.async_copy(w_hbm.at[pl.ds(r, CROWS), :], bufs[2], sem)

    def wait(bufs, sem, k):
        r = row0 + k * CROWS
        pltpu.make_async_copy(p_hbm.at[pl.ds(r, CROWS), :], bufs[0], sem).wait()
        pltpu.make_async_copy(t_hbm.at[pl.ds(r, CROWS), :], bufs[1], sem).wait()
        pltpu.make_async_copy(w_hbm.at[pl.ds(r, CROWS), :], bufs[2], sem).wait()

    start(bufs0, sem0, 0)

    def outer(k, carry):
        start(bufs1, sem1, 2 * k + 1)
        wait(bufs0, sem0, 2 * k)
        carry = compute(bufs0, carry)
        start(bufs0, sem0, 2 * k + 2)
        wait(bufs1, sem1, 2 * k + 1)
        carry = compute(bufs1, carry)
        return carry

    carry = lax.fori_loop(0, nchunk // 2 - 1, outer, carry)
    start(bufs1, sem1, nchunk - 1)
    wait(bufs0, sem0, nchunk - 2)
    carry = compute(bufs0, carry)
    wait(bufs1, sem1, nchunk - 1)
    carry = compute(bufs1, carry)
    return carry


def _load3(bufs, i):
    r = lax.shift_right_logical(i, 10)
    c = lax.bitwise_and(i, COLS - 1)
    pb, tb, wb = bufs
    return (pb[r, pl.ds(c, L)], tb[r, pl.ds(c, L)], wb[r, pl.ds(c, L)])


def _pass1_body(nchunk, p_hbm, t_hbm, w_hbm, out_hbm,
                pb0, tb0, wb0, pb1, tb1, wb1, accb, sem0, sem1):
    def compute(bufs, acc):
        def inner(j, acc):
            a_i, a_p, a_t, a_v = acc
            for u in range(UNROLL):
                p, t, w = _load3(bufs, j * (L * UNROLL) + u * L)
                a_i = a_i + p * t
                a_p = a_p + p
                a_t = a_t + t
                a_v = a_v + jnp.where(w > 0.0, 1.0, 0.0).astype(jnp.float32)
            return (a_i, a_p, a_t, a_v)

        return lax.fori_loop(0, CHUNK // (L * UNROLL), inner, acc)

    z = jnp.zeros((L,), jnp.float32)
    acc = _stream_loop(p_hbm, t_hbm, w_hbm, (pb0, tb0, wb0), (pb1, tb1, wb1),
                       sem0, sem1, nchunk, compute, (z, z, z, z))
    for i in range(4):
        accb[i, :] = acc[i]
    pltpu.sync_copy(accb, out_hbm.at[_wid()])


def _pass2_body(nchunk, p_hbm, t_hbm, w_hbm, c_hbm, out_hbm,
                pb0, tb0, wb0, pb1, tb1, wb1, cb, hist, sem0, sem1):
    pltpu.sync_copy(c_hbm, cb)
    c = cb[...]
    for i in range(2 * BINS):
        hist[pl.ds(i * L, L)] = jnp.zeros((L,), jnp.float32)
    lane = lax.iota(jnp.int32, L)
    ones = jnp.ones((L,), jnp.float32)

    def compute(bufs, carry):
        @plsc.parallel_loop(0, CHUNK, L, unroll=8)
        def body(i):
            p, t, w = _load3(bufs, i)
            g = jnp.abs(c * p - t)
            b = jnp.minimum((g * 10.0).astype(jnp.int32), BINS - 1)
            m = (g < THRESH) & (w > 0.0)
            slot = b * L + lane
            plsc.addupdate_scatter(hist, [slot], ones, mask=m)
            plsc.addupdate_scatter(hist, [slot + BINS * L], p * t, mask=m)

        return carry

    _stream_loop(p_hbm, t_hbm, w_hbm, (pb0, tb0, wb0), (pb1, tb1, wb1),
                 sem0, sem1, nchunk, compute, 0)
    pltpu.sync_copy(hist, out_hbm.at[_wid()])


@functools.cache
def _build(nrows):
    assert nrows % (NW * CROWS * 2) == 0, nrows
    nchunk = nrows // (NW * CROWS)
    stream_bufs = [pltpu.VMEM((CROWS, COLS), jnp.float32) for _ in range(6)]
    params = pltpu.CompilerParams(needs_layout_passes=False)

    pass1 = pl.kernel(
        functools.partial(_pass1_body, nchunk),
        out_type=jax.ShapeDtypeStruct((NW, 4, L), jnp.float32),
        mesh=_mesh(),
        compiler_params=params,
        scratch_types=stream_bufs
        + [pltpu.VMEM((4, L), jnp.float32),
           pltpu.SemaphoreType.DMA, pltpu.SemaphoreType.DMA],
    )
    pass2 = pl.kernel(
        functools.partial(_pass2_body, nchunk),
        out_type=jax.ShapeDtypeStruct((NW, 2 * BINS * L), jnp.float32),
        mesh=_mesh(),
        compiler_params=params,
        scratch_types=stream_bufs
        + [pltpu.VMEM((L,), jnp.float32),
           pltpu.VMEM((2 * BINS * L,), jnp.float32),
           pltpu.SemaphoreType.DMA, pltpu.SemaphoreType.DMA],
    )
    return pass1, pass2


def kernel(pred, target, label_weight):
    p = pred
    t = target.astype(jnp.float32)
    w = label_weight.astype(jnp.float32)
    pass1, pass2 = _build(p.shape[0])

    part1 = pass1(p, t, w)
    s = jnp.sum(part1, axis=(0, 2))
    i_sum, sum_p, sum_t, n_valid = s[0], s[1], s[2], s[3]
    big_s = sum_p + sum_t
    tot = jnp.maximum(n_valid, 1.0)
    cvec = jnp.full((L,), 2.0 * i_sum / big_s, jnp.float32)

    part2 = pass2(p, t, w, cvec)
    h = jnp.sum(part2.reshape(NW, 2 * BINS, L), axis=(0, 2))
    counts = h[:BINS]
    ptb = h[BINS:]
    n = jnp.sum((counts > 0).astype(jnp.float32))
    contrib = jnp.where(counts > 0, tot / jnp.maximum(counts, 1.0), 0.0) * ptb
    loss = 1.0 - (2.0 * jnp.sum(contrib) / jnp.maximum(n, 1.0)) / big_s
    return loss


# speedup vs baseline: 105.8169x; 1.1049x over previous
import functools

import numpy as np
import jax
import jax.numpy as jnp
from jax import lax
from jax.experimental import pallas as pl
from jax.experimental.pallas import tpu as pltpu
from jax.experimental.pallas import tpu_sc as plsc

NC = 2
NS = 16
L = 16
NW = NC * NS
BINS = 10
NB = BINS + 1
COLS = 1024
CROWS = 16
CHUNK = CROWS * COLS
UNROLL = 4
THRESH10 = float(np.float32(10.0) * (np.float32(1.0) + np.float32(1e-6)))


def _wid():
    return lax.axis_index("s") * NC + lax.axis_index("c")


def _mesh():
    return plsc.VectorSubcoreMesh(
        core_axis_name="c", subcore_axis_name="s", num_cores=NC, num_subcores=NS
    )


def _stream_loop(arrays, bufs0, bufs1, sem0, sem1, nchunk, compute, carry):
    row0 = _wid() * (nchunk * CROWS)

    def start(bufs, sem, k):
        r = row0 + k * CROWS
        for a, b in zip(arrays, bufs):
            pltpu.async_copy(a.at[pl.ds(r, CROWS), :], b, sem)

    def wait(bufs, sem, k):
        r = row0 + k * CROWS
        for a, b in zip(arrays, bufs):
            pltpu.make_async_copy(a.at[pl.ds(r, CROWS), :], b, sem).wait()

    start(bufs0, sem0, 0)

    def outer(k, carry):
        start(bufs1, sem1, 2 * k + 1)
        wait(bufs0, sem0, 2 * k)
        carry = compute(bufs0, carry)
        start(bufs0, sem0, 2 * k + 2)
        wait(bufs1, sem1, 2 * k + 1)
        carry = compute(bufs1, carry)
        return carry

    carry = lax.fori_loop(0, nchunk // 2 - 1, outer, carry)
    start(bufs1, sem1, nchunk - 1)
    wait(bufs0, sem0, nchunk - 2)
    carry = compute(bufs0, carry)
    wait(bufs1, sem1, nchunk - 1)
    carry = compute(bufs1, carry)
    return carry


def _group(buf, i):
    return buf[lax.shift_right_logical(i, 10), pl.ds(lax.bitwise_and(i, COLS - 1), L)]


def _pass1_body(nchunk, p_hbm, t_hbm, out_hbm, pb0, tb0, pb1, tb1, accb, sem0, sem1):
    def compute(bufs, acc):
        pb, tb = bufs

        def inner(j, acc):
            acc = list(acc)
            for u in range(UNROLL):
                o = j * (L * UNROLL) + u * L
                p = _group(pb, o)
                t = _group(tb, o)
                a_i, a_p, a_t = acc[u]
                acc[u] = (a_i + p * t, a_p + p, a_t + t)
            return tuple(acc)

        return lax.fori_loop(0, CHUNK // (L * UNROLL), inner, acc)

    z = jnp.zeros((L,), jnp.float32)
    acc = _stream_loop((p_hbm, t_hbm), (pb0, tb0), (pb1, tb1),
                       sem0, sem1, nchunk, compute, ((z, z, z),) * UNROLL)
    for i in range(3):
        v = acc[0][i]
        for u in range(1, UNROLL):
            v = v + acc[u][i]
        accb[i, :] = v
    accb[3, :] = jnp.zeros((L,), jnp.float32)
    pltpu.sync_copy(accb, out_hbm.at[_wid()])


def _pass2_body(nchunk, p_hbm, t_hbm, w_hbm, part1_hbm, out_hbm,
                pb0, tb0, wb0, pb1, tb1, wb1, partb, hist, sem0, sem1):
    pltpu.sync_copy(part1_hbm, partb)
    z = jnp.zeros((L,), jnp.float32)
    a_i, a_p, a_t = z, z, z
    for w in range(NW):
        a_i = a_i + partb[w, 0, :]
        a_p = a_p + partb[w, 1, :]
        a_t = a_t + partb[w, 2, :]
    i_sum = jnp.sum(a_i)
    big_s = jnp.sum(a_p + a_t)
    c10 = (20.0 * jnp.full((L,), i_sum, jnp.float32)) / jnp.full((L,), big_s, jnp.float32)

    for i in range(2 * NB):
        hist[pl.ds(i * L, L)] = jnp.zeros((L,), jnp.float32)
    lane = lax.iota(jnp.int32, L)
    ones = jnp.ones((L,), jnp.float32)

    def compute(bufs, carry):
        pb, tb, wb = bufs

        @plsc.parallel_loop(0, CHUNK, L, unroll=8)
        def body(i):
            p = _group(pb, i)
            t = _group(tb, i)
            w = _group(wb, i)
            g10 = jnp.abs(c10 * p - 10.0 * t)
            b = jnp.where(g10 < THRESH10,
                          jnp.minimum(g10.astype(jnp.int32), BINS - 1), BINS)
            m = w > 0.0
            slot = b * L + lane
            plsc.addupdate_scatter(hist, [slot], ones, mask=m)
            plsc.addupdate_scatter(hist, [slot + NB * L], p * t, mask=m)

        return carry

    _stream_loop((p_hbm, t_hbm, w_hbm), (pb0, tb0, wb0), (pb1, tb1, wb1),
                 sem0, sem1, nchunk, compute, 0)
    pltpu.sync_copy(hist, out_hbm.at[_wid()])


@functools.cache
def _build(nrows):
    assert nrows % (NW * CROWS * 2) == 0, nrows
    nchunk = nrows // (NW * CROWS)
    buf = lambda: pltpu.VMEM((CROWS, COLS), jnp.float32)
    params = pltpu.CompilerParams(needs_layout_passes=False)

    pass1 = pl.kernel(
        functools.partial(_pass1_body, nchunk),
        out_type=jax.ShapeDtypeStruct((NW, 4, L), jnp.float32),
        mesh=_mesh(),
        compiler_params=params,
        scratch_types=[buf() for _ in range(4)]
        + [pltpu.VMEM((4, L), jnp.float32),
           pltpu.SemaphoreType.DMA, pltpu.SemaphoreType.DMA],
    )
    pass2 = pl.kernel(
        functools.partial(_pass2_body, nchunk),
        out_type=jax.ShapeDtypeStruct((NW, 2 * NB * L), jnp.float32),
        mesh=_mesh(),
        compiler_params=params,
        scratch_types=[buf() for _ in range(6)]
        + [pltpu.VMEM((NW, 4, L), jnp.float32),
           pltpu.VMEM((2 * NB * L,), jnp.float32),
           pltpu.SemaphoreType.DMA, pltpu.SemaphoreType.DMA],
    )
    return pass1, pass2


def kernel(pred, target, label_weight):
    p = pred
    t = target.astype(jnp.float32)
    w = label_weight.astype(jnp.float32)
    pass1, pass2 = _build(p.shape[0])

    part1 = pass1(p, t)
    part2 = pass2(p, t, w, part1)

    s = jnp.sum(part1, axis=(0, 2))
    big_s = s[1] + s[2]
    h = jnp.sum(part2.reshape(NW, 2 * NB, L), axis=(0, 2))
    counts = h[:BINS]
    tot = jnp.maximum(jnp.sum(h[:NB]), 1.0)
    ptb = h[NB:NB + BINS]
    n = jnp.sum((counts > 0).astype(jnp.float32))
    contrib = jnp.where(counts > 0, tot / jnp.maximum(counts, 1.0), 0.0) * ptb
    loss = 1.0 - (2.0 * jnp.sum(contrib) / jnp.maximum(n, 1.0)) / big_s
    return loss
